# baseline (speedup 1.0000x reference)
import functools

import jax
import jax.numpy as jnp
from jax import lax
from jax.experimental import pallas as pl
from jax.experimental.pallas import tpu as pltpu
from jax.experimental.pallas import tpu_sc as plsc

N = 10000
D = 128
E = 320000

NC = 2
NS = 16
NW = NC * NS
EPW = E // NW
CH = 16
NCHUNK = EPW // CH
NACC = 10240
RPT = NACC // NS

BM = 2000


KB = 16
LA = 12
LS = 4
KI = 25


def _sc_agg_body(ht_hbm, ei_hbm, out_hbm,
                 acc_sh, src_r, dst_r, rows_v, gsem, ssem, isem):
    c = lax.axis_index("c")
    s = lax.axis_index("s")
    w = c * NS + s

    def src_off(g):
        return pl.multiple_of(w * EPW + g * CH, 8)

    def dst_off(g):
        return pl.multiple_of(E + w * EPW + g * CH, 8)

    def fire_idx(g):
        sl = lax.rem(g, KI)
        pltpu.async_copy(ei_hbm.at[pl.ds(src_off(g), CH)], src_r.at[sl], isem)
        pltpu.async_copy(ei_hbm.at[pl.ds(dst_off(g), CH)], dst_r.at[sl], isem)

    def wait_idx_and_fire_gather(g):
        sl = lax.rem(g, KI)
        pltpu.make_async_copy(ei_hbm.at[pl.ds(src_off(g), CH)],
                              src_r.at[sl], isem).wait()
        pltpu.make_async_copy(ei_hbm.at[pl.ds(dst_off(g), CH)],
                              dst_r.at[sl], isem).wait()
        pltpu.async_copy(ht_hbm.at[src_r.at[sl]], rows_v.at[lax.rem(g, KB)],
                         gsem)

    def step(j, drain_scatter, fire_next_idx, fire_gather):
        b = lax.rem(j, KB)
        sl = lax.rem(j, KI)
        pltpu.make_async_copy(ht_hbm.at[src_r.at[sl]], rows_v.at[b],
                              gsem).wait()
        if drain_scatter:
            pltpu.make_async_copy(rows_v.at[b], acc_sh.at[dst_r.at[sl]],
                                  ssem).wait()
        pltpu.async_copy(rows_v.at[b], acc_sh.at[dst_r.at[sl]], ssem,
                         add=True)
        if fire_next_idx:
            fire_idx(j + KI - LA)
        if fire_gather:
            wait_idx_and_fire_gather(j + LA)

    for g in range(KI):
        fire_idx(g)
    for g in range(LA):
        wait_idx_and_fire_gather(g)

    def zbody(i, carry):
        r = i // 8
        k = (i % 8) * 16
        rows_v[LA, r, pl.ds(k, 16)] = jnp.zeros((16,), jnp.float32)
        return carry
    lax.fori_loop(0, CH * 8, zbody, 0)
    for t in range(RPT // CH):
        pltpu.sync_copy(rows_v.at[LA], acc_sh.at[pl.ds(s * RPT + t * CH, CH)])
    plsc.subcore_barrier()

    for j in range(LS):
        step(j, drain_scatter=False, fire_next_idx=False, fire_gather=True)
    for j in range(LS, LA):
        step(j, drain_scatter=True, fire_next_idx=False, fire_gather=True)

    def body_full(j, carry):
        step(j, drain_scatter=True, fire_next_idx=True, fire_gather=True)
        return carry
    lax.fori_loop(LA, NCHUNK - KI + LA, body_full, 0)

    def body_noidx(j, carry):
        step(j, drain_scatter=True, fire_next_idx=False, fire_gather=True)
        return carry
    lax.fori_loop(NCHUNK - KI + LA, NCHUNK - LA, body_noidx, 0)

    def body_nogather(j, carry):
        step(j, drain_scatter=True, fire_next_idx=False, fire_gather=False)
        return carry
    lax.fori_loop(NCHUNK - LA, NCHUNK, body_nogather, 0)

    for _ in range(LS):
        pltpu.make_async_copy(rows_v.at[0], acc_sh.at[dst_r.at[0]],
                              ssem).wait()

    plsc.subcore_barrier()
    pltpu.sync_copy(acc_sh.at[pl.ds(s * RPT, RPT)],
                    out_hbm.at[c, pl.ds(s * RPT, RPT)])


def _sc_agg(ht, ei):
    mesh = plsc.VectorSubcoreMesh(core_axis_name="c", subcore_axis_name="s",
                                  num_cores=NC, num_subcores=NS)
    fn = pl.kernel(
        _sc_agg_body,
        out_type=jax.ShapeDtypeStruct((NC, NACC, D), jnp.float32),
        mesh=mesh,
        scratch_types=[
            pltpu.VMEM_SHARED((NACC, D), jnp.float32),
            pltpu.VMEM((KI, CH), jnp.int32),
            pltpu.VMEM((KI, CH), jnp.int32),
            pltpu.VMEM((KB, CH, D), jnp.float32),
            pltpu.SemaphoreType.DMA,
            pltpu.SemaphoreType.DMA,
            pltpu.SemaphoreType.DMA,
        ],
    )
    return fn(ht, ei)


def _mm_body(h_ref, w_ref, out_ref):
    out_ref[...] = jnp.dot(h_ref[...], w_ref[...],
                           preferred_element_type=jnp.float32)


def _mm(h, w):
    return pl.pallas_call(
        _mm_body,
        grid=(N // BM,),
        in_specs=[
            pl.BlockSpec((BM, D), lambda i: (i, 0)),
            pl.BlockSpec((D, D), lambda i: (0, 0)),
        ],
        out_specs=pl.BlockSpec((BM, D), lambda i: (i, 0)),
        out_shape=jax.ShapeDtypeStruct((N, D), jnp.float32),
    )(h, w)


def _combine(part_ref, self_ref):
    y = part_ref[0] + part_ref[1] + self_ref[...]
    nrm = jnp.sqrt(jnp.sum(y * y, axis=1, keepdims=True))
    y = y / jnp.maximum(nrm, 1e-12)
    return jnp.maximum(y, 0.0)


def _k2_body(part_ref, self_ref, wr_ref, h_ref, ht_ref):
    y = _combine(part_ref, self_ref)
    h_ref[...] = y
    ht_ref[...] = jnp.dot(y, wr_ref[...], preferred_element_type=jnp.float32)


def _k2(part, selfv, wr):
    return pl.pallas_call(
        _k2_body,
        grid=(N // BM,),
        in_specs=[
            pl.BlockSpec((NC, BM, D), lambda i: (0, i, 0)),
            pl.BlockSpec((BM, D), lambda i: (i, 0)),
            pl.BlockSpec((D, D), lambda i: (0, 0)),
        ],
        out_specs=[
            pl.BlockSpec((BM, D), lambda i: (i, 0)),
            pl.BlockSpec((BM, D), lambda i: (i, 0)),
        ],
        out_shape=[
            jax.ShapeDtypeStruct((N, D), jnp.float32),
            jax.ShapeDtypeStruct((N, D), jnp.float32),
        ],
    )(part, selfv, wr)


def _k3_body(part_ref, self_ref, w3_ref, b3_ref, w4_ref, b4_ref, out_ref):
    y = _combine(part_ref, self_ref)
    z = jnp.dot(y, w3_ref[...], preferred_element_type=jnp.float32) + b3_ref[...]
    z = jnp.dot(z, w4_ref[...], preferred_element_type=jnp.float32) + b4_ref[...]
    m = jnp.max(z, axis=1, keepdims=True)
    lse = jnp.log(jnp.sum(jnp.exp(z - m), axis=1, keepdims=True))
    out_ref[...] = z - m - lse


def _k3(part, selfv, w3, b3, w4, b4):
    return pl.pallas_call(
        _k3_body,
        grid=(N // BM,),
        in_specs=[
            pl.BlockSpec((NC, BM, D), lambda i: (0, i, 0)),
            pl.BlockSpec((BM, D), lambda i: (i, 0)),
            pl.BlockSpec((D, D), lambda i: (0, 0)),
            pl.BlockSpec((1, D), lambda i: (0, 0)),
            pl.BlockSpec((D, D), lambda i: (0, 0)),
            pl.BlockSpec((1, D), lambda i: (0, 0)),
        ],
        out_specs=pl.BlockSpec((BM, D), lambda i: (i, 0)),
        out_shape=jax.ShapeDtypeStruct((N, D), jnp.float32),
    )(part, selfv, w3, b3, w4, b4)


@jax.jit
def kernel(x, edge_index, emb, Wl1, Wr1, Wl2, Wr2, W3, b3, W4, b4):
    del x
    ei = edge_index.astype(jnp.int32).reshape(2 * E)
    b3r = b3.reshape(1, D)
    b4r = b4.reshape(1, D)

    ht1 = _mm(emb, Wr1)
    part1 = _sc_agg(ht1, ei)
    self1 = _mm(emb, Wl1)
    h1, ht2 = _k2(part1, self1, Wr2)
    part2 = _sc_agg(ht2, ei)
    self2 = _mm(h1, Wl2)
    return _k3(part2, self2, W3, b3r, W4, b4r)


# speedup vs baseline: 1.6693x; 1.6693x over previous
import functools

import jax
import jax.numpy as jnp
from jax import lax
from jax.experimental import pallas as pl
from jax.experimental.pallas import tpu as pltpu
from jax.experimental.pallas import tpu_sc as plsc

N = 10000
D = 128
E = 320000

NC = 2
NS = 16
NW = NC * NS
EPW = E // NW
CH = 40
NCHUNK = EPW // CH
NACC = 10240
RPT = NACC // NS

BM = 2000


KB = 8
LA = 5
LS = 3
KI = 16


def _sc_agg_body(ht_hbm, ei_hbm, out_hbm,
                 acc_sh, src_r, dst_r, rows_v, gsem, ssem, isem):
    c = lax.axis_index("c")
    s = lax.axis_index("s")
    w = c * NS + s

    def src_off(g):
        return pl.multiple_of(w * EPW + g * CH, 8)

    def dst_off(g):
        return pl.multiple_of(E + w * EPW + g * CH, 8)

    def fire_idx(g):
        sl = lax.rem(g, KI)
        pltpu.async_copy(ei_hbm.at[pl.ds(src_off(g), CH)], src_r.at[sl], isem)
        pltpu.async_copy(ei_hbm.at[pl.ds(dst_off(g), CH)], dst_r.at[sl], isem)

    def wait_idx_and_fire_gather(g):
        sl = lax.rem(g, KI)
        pltpu.make_async_copy(ei_hbm.at[pl.ds(src_off(g), CH)],
                              src_r.at[sl], isem).wait()
        pltpu.make_async_copy(ei_hbm.at[pl.ds(dst_off(g), CH)],
                              dst_r.at[sl], isem).wait()
        pltpu.async_copy(ht_hbm.at[src_r.at[sl]], rows_v.at[lax.rem(g, KB)],
                         gsem)

    def step(j, drain_scatter, fire_next_idx, fire_gather):
        b = lax.rem(j, KB)
        sl = lax.rem(j, KI)
        pltpu.make_async_copy(ht_hbm.at[src_r.at[sl]], rows_v.at[b],
                              gsem).wait()
        if drain_scatter:
            pltpu.make_async_copy(rows_v.at[b], acc_sh.at[dst_r.at[sl]],
                                  ssem).wait()
        pltpu.async_copy(rows_v.at[b], acc_sh.at[dst_r.at[sl]], ssem,
                         add=True)
        if fire_next_idx:
            fire_idx(j + KI - LA)
        if fire_gather:
            wait_idx_and_fire_gather(j + LA)

    for g in range(KI):
        fire_idx(g)
    for g in range(LA):
        wait_idx_and_fire_gather(g)

    def zbody(i, carry):
        r = i // 8
        k = (i % 8) * 16
        rows_v[LA, r, pl.ds(k, 16)] = jnp.zeros((16,), jnp.float32)
        return carry
    lax.fori_loop(0, CH * 8, zbody, 0)
    for t in range(RPT // CH):
        pltpu.sync_copy(rows_v.at[LA], acc_sh.at[pl.ds(s * RPT + t * CH, CH)])
    plsc.subcore_barrier()

    for j in range(LS):
        step(j, drain_scatter=False, fire_next_idx=False, fire_gather=True)
    for j in range(LS, LA):
        step(j, drain_scatter=True, fire_next_idx=False, fire_gather=True)

    def body_full(j, carry):
        step(j, drain_scatter=True, fire_next_idx=True, fire_gather=True)
        return carry
    lax.fori_loop(LA, NCHUNK - KI + LA, body_full, 0)

    def body_noidx(j, carry):
        step(j, drain_scatter=True, fire_next_idx=False, fire_gather=True)
        return carry
    lax.fori_loop(NCHUNK - KI + LA, NCHUNK - LA, body_noidx, 0)

    def body_nogather(j, carry):
        step(j, drain_scatter=True, fire_next_idx=False, fire_gather=False)
        return carry
    lax.fori_loop(NCHUNK - LA, NCHUNK, body_nogather, 0)

    for _ in range(LS):
        pltpu.make_async_copy(rows_v.at[0], acc_sh.at[dst_r.at[0]],
                              ssem).wait()

    plsc.subcore_barrier()
    pltpu.sync_copy(acc_sh.at[pl.ds(s * RPT, RPT)],
                    out_hbm.at[c, pl.ds(s * RPT, RPT)])


def _sc_agg(ht, ei):
    mesh = plsc.VectorSubcoreMesh(core_axis_name="c", subcore_axis_name="s",
                                  num_cores=NC, num_subcores=NS)
    fn = pl.kernel(
        _sc_agg_body,
        out_type=jax.ShapeDtypeStruct((NC, NACC, D), jnp.float32),
        mesh=mesh,
        scratch_types=[
            pltpu.VMEM_SHARED((NACC, D), jnp.float32),
            pltpu.VMEM((KI, CH), jnp.int32),
            pltpu.VMEM((KI, CH), jnp.int32),
            pltpu.VMEM((KB, CH, D), jnp.float32),
            pltpu.SemaphoreType.DMA,
            pltpu.SemaphoreType.DMA,
            pltpu.SemaphoreType.DMA,
        ],
    )
    return fn(ht, ei)


def _mm_body(h_ref, w_ref, out_ref):
    out_ref[...] = jnp.dot(h_ref[...], w_ref[...],
                           preferred_element_type=jnp.float32)


def _mm(h, w):
    return pl.pallas_call(
        _mm_body,
        grid=(N // BM,),
        in_specs=[
            pl.BlockSpec((BM, D), lambda i: (i, 0)),
            pl.BlockSpec((D, D), lambda i: (0, 0)),
        ],
        out_specs=pl.BlockSpec((BM, D), lambda i: (i, 0)),
        out_shape=jax.ShapeDtypeStruct((N, D), jnp.float32),
    )(h, w)


def _combine(part_ref, self_ref):
    y = part_ref[0] + part_ref[1] + self_ref[...]
    nrm = jnp.sqrt(jnp.sum(y * y, axis=1, keepdims=True))
    y = y / jnp.maximum(nrm, 1e-12)
    return jnp.maximum(y, 0.0)


def _k2_body(part_ref, self_ref, wr_ref, h_ref, ht_ref):
    y = _combine(part_ref, self_ref)
    h_ref[...] = y
    ht_ref[...] = jnp.dot(y, wr_ref[...], preferred_element_type=jnp.float32)


def _k2(part, selfv, wr):
    return pl.pallas_call(
        _k2_body,
        grid=(N // BM,),
        in_specs=[
            pl.BlockSpec((NC, BM, D), lambda i: (0, i, 0)),
            pl.BlockSpec((BM, D), lambda i: (i, 0)),
            pl.BlockSpec((D, D), lambda i: (0, 0)),
        ],
        out_specs=[
            pl.BlockSpec((BM, D), lambda i: (i, 0)),
            pl.BlockSpec((BM, D), lambda i: (i, 0)),
        ],
        out_shape=[
            jax.ShapeDtypeStruct((N, D), jnp.float32),
            jax.ShapeDtypeStruct((N, D), jnp.float32),
        ],
    )(part, selfv, wr)


def _k3_body(part_ref, self_ref, w3_ref, b3_ref, w4_ref, b4_ref, out_ref):
    y = _combine(part_ref, self_ref)
    z = jnp.dot(y, w3_ref[...], preferred_element_type=jnp.float32) + b3_ref[...]
    z = jnp.dot(z, w4_ref[...], preferred_element_type=jnp.float32) + b4_ref[...]
    m = jnp.max(z, axis=1, keepdims=True)
    lse = jnp.log(jnp.sum(jnp.exp(z - m), axis=1, keepdims=True))
    out_ref[...] = z - m - lse


def _k3(part, selfv, w3, b3, w4, b4):
    return pl.pallas_call(
        _k3_body,
        grid=(N // BM,),
        in_specs=[
            pl.BlockSpec((NC, BM, D), lambda i: (0, i, 0)),
            pl.BlockSpec((BM, D), lambda i: (i, 0)),
            pl.BlockSpec((D, D), lambda i: (0, 0)),
            pl.BlockSpec((1, D), lambda i: (0, 0)),
            pl.BlockSpec((D, D), lambda i: (0, 0)),
            pl.BlockSpec((1, D), lambda i: (0, 0)),
        ],
        out_specs=pl.BlockSpec((BM, D), lambda i: (i, 0)),
        out_shape=jax.ShapeDtypeStruct((N, D), jnp.float32),
    )(part, selfv, w3, b3, w4, b4)


@jax.jit
def kernel(x, edge_index, emb, Wl1, Wr1, Wl2, Wr2, W3, b3, W4, b4):
    del x
    ei = edge_index.astype(jnp.int32).reshape(2 * E)
    b3r = b3.reshape(1, D)
    b4r = b4.reshape(1, D)

    ht1 = _mm(emb, Wr1)
    part1 = _sc_agg(ht1, ei)
    self1 = _mm(emb, Wl1)
    h1, ht2 = _k2(part1, self1, Wr2)
    part2 = _sc_agg(ht2, ei)
    self2 = _mm(h1, Wl2)
    return _k3(part2, self2, W3, b3r, W4, b4r)


# speedup vs baseline: 1.6737x; 1.0027x over previous
import functools

import jax
import jax.numpy as jnp
from jax import lax
from jax.experimental import pallas as pl
from jax.experimental.pallas import tpu as pltpu
from jax.experimental.pallas import tpu_sc as plsc

N = 10000
D = 128
E = 320000

NC = 2
NS = 16
NW = NC * NS
EPW = E // NW
CH = 40
NCHUNK = EPW // CH
NACC = 10240
RPT = NACC // NS

BM = 2000


KB = 8
LA = 6
LS = 2
KI = 16


def _sc_agg_body(ht_hbm, ei_hbm, out_hbm,
                 acc_sh, src_r, dst_r, rows_v, gsem, ssem, isem):
    c = lax.axis_index("c")
    s = lax.axis_index("s")
    w = c * NS + s

    def src_off(g):
        return pl.multiple_of(w * EPW + g * CH, 8)

    def dst_off(g):
        return pl.multiple_of(E + w * EPW + g * CH, 8)

    def fire_idx(g):
        sl = lax.rem(g, KI)
        pltpu.async_copy(ei_hbm.at[pl.ds(src_off(g), CH)], src_r.at[sl], isem)
        pltpu.async_copy(ei_hbm.at[pl.ds(dst_off(g), CH)], dst_r.at[sl], isem)

    def wait_idx_and_fire_gather(g):
        sl = lax.rem(g, KI)
        pltpu.make_async_copy(ei_hbm.at[pl.ds(src_off(g), CH)],
                              src_r.at[sl], isem).wait()
        pltpu.make_async_copy(ei_hbm.at[pl.ds(dst_off(g), CH)],
                              dst_r.at[sl], isem).wait()
        pltpu.async_copy(ht_hbm.at[src_r.at[sl]], rows_v.at[lax.rem(g, KB)],
                         gsem)

    def step(j, drain_scatter, fire_next_idx, fire_gather):
        b = lax.rem(j, KB)
        sl = lax.rem(j, KI)
        pltpu.make_async_copy(ht_hbm.at[src_r.at[sl]], rows_v.at[b],
                              gsem).wait()
        if drain_scatter:
            pltpu.make_async_copy(rows_v.at[b], acc_sh.at[dst_r.at[sl]],
                                  ssem).wait()
        pltpu.async_copy(rows_v.at[b], acc_sh.at[dst_r.at[sl]], ssem,
                         add=True)
        if fire_next_idx:
            fire_idx(j + KI - LA)
        if fire_gather:
            wait_idx_and_fire_gather(j + LA)

    for g in range(KI):
        fire_idx(g)
    for g in range(LA):
        wait_idx_and_fire_gather(g)

    def zbody(i, carry):
        r = i // 8
        k = (i % 8) * 16
        rows_v[LA, r, pl.ds(k, 16)] = jnp.zeros((16,), jnp.float32)
        return carry
    lax.fori_loop(0, CH * 8, zbody, 0)
    for t in range(RPT // CH):
        pltpu.sync_copy(rows_v.at[LA], acc_sh.at[pl.ds(s * RPT + t * CH, CH)])
    plsc.subcore_barrier()

    for j in range(LS):
        step(j, drain_scatter=False, fire_next_idx=False, fire_gather=True)
    for j in range(LS, LA):
        step(j, drain_scatter=True, fire_next_idx=False, fire_gather=True)

    def body_full(j, carry):
        step(j, drain_scatter=True, fire_next_idx=True, fire_gather=True)
        return carry
    lax.fori_loop(LA, NCHUNK - KI + LA, body_full, 0)

    def body_noidx(j, carry):
        step(j, drain_scatter=True, fire_next_idx=False, fire_gather=True)
        return carry
    lax.fori_loop(NCHUNK - KI + LA, NCHUNK - LA, body_noidx, 0)

    def body_nogather(j, carry):
        step(j, drain_scatter=True, fire_next_idx=False, fire_gather=False)
        return carry
    lax.fori_loop(NCHUNK - LA, NCHUNK, body_nogather, 0)

    for _ in range(LS):
        pltpu.make_async_copy(rows_v.at[0], acc_sh.at[dst_r.at[0]],
                              ssem).wait()

    plsc.subcore_barrier()
    pltpu.sync_copy(acc_sh.at[pl.ds(s * RPT, RPT)],
                    out_hbm.at[c, pl.ds(s * RPT, RPT)])


def _sc_agg(ht, ei):
    mesh = plsc.VectorSubcoreMesh(core_axis_name="c", subcore_axis_name="s",
                                  num_cores=NC, num_subcores=NS)
    fn = pl.kernel(
        _sc_agg_body,
        out_type=jax.ShapeDtypeStruct((NC, NACC, D), jnp.float32),
        mesh=mesh,
        scratch_types=[
            pltpu.VMEM_SHARED((NACC, D), jnp.float32),
            pltpu.VMEM((KI, CH), jnp.int32),
            pltpu.VMEM((KI, CH), jnp.int32),
            pltpu.VMEM((KB, CH, D), jnp.float32),
            pltpu.SemaphoreType.DMA,
            pltpu.SemaphoreType.DMA,
            pltpu.SemaphoreType.DMA,
        ],
    )
    return fn(ht, ei)


def _mm_body(h_ref, w_ref, out_ref):
    out_ref[...] = jnp.dot(h_ref[...], w_ref[...],
                           preferred_element_type=jnp.float32)


def _mm(h, w):
    return pl.pallas_call(
        _mm_body,
        grid=(N // BM,),
        in_specs=[
            pl.BlockSpec((BM, D), lambda i: (i, 0)),
            pl.BlockSpec((D, D), lambda i: (0, 0)),
        ],
        out_specs=pl.BlockSpec((BM, D), lambda i: (i, 0)),
        out_shape=jax.ShapeDtypeStruct((N, D), jnp.float32),
    )(h, w)


def _combine(part_ref, self_ref):
    y = part_ref[0] + part_ref[1] + self_ref[...]
    nrm = jnp.sqrt(jnp.sum(y * y, axis=1, keepdims=True))
    y = y / jnp.maximum(nrm, 1e-12)
    return jnp.maximum(y, 0.0)


def _k2_body(part_ref, self_ref, wr_ref, h_ref, ht_ref):
    y = _combine(part_ref, self_ref)
    h_ref[...] = y
    ht_ref[...] = jnp.dot(y, wr_ref[...], preferred_element_type=jnp.float32)


def _k2(part, selfv, wr):
    return pl.pallas_call(
        _k2_body,
        grid=(N // BM,),
        in_specs=[
            pl.BlockSpec((NC, BM, D), lambda i: (0, i, 0)),
            pl.BlockSpec((BM, D), lambda i: (i, 0)),
            pl.BlockSpec((D, D), lambda i: (0, 0)),
        ],
        out_specs=[
            pl.BlockSpec((BM, D), lambda i: (i, 0)),
            pl.BlockSpec((BM, D), lambda i: (i, 0)),
        ],
        out_shape=[
            jax.ShapeDtypeStruct((N, D), jnp.float32),
            jax.ShapeDtypeStruct((N, D), jnp.float32),
        ],
    )(part, selfv, wr)


def _k3_body(part_ref, self_ref, w3_ref, b3_ref, w4_ref, b4_ref, out_ref):
    y = _combine(part_ref, self_ref)
    z = jnp.dot(y, w3_ref[...], preferred_element_type=jnp.float32) + b3_ref[...]
    z = jnp.dot(z, w4_ref[...], preferred_element_type=jnp.float32) + b4_ref[...]
    m = jnp.max(z, axis=1, keepdims=True)
    lse = jnp.log(jnp.sum(jnp.exp(z - m), axis=1, keepdims=True))
    out_ref[...] = z - m - lse


def _k3(part, selfv, w3, b3, w4, b4):
    return pl.pallas_call(
        _k3_body,
        grid=(N // BM,),
        in_specs=[
            pl.BlockSpec((NC, BM, D), lambda i: (0, i, 0)),
            pl.BlockSpec((BM, D), lambda i: (i, 0)),
            pl.BlockSpec((D, D), lambda i: (0, 0)),
            pl.BlockSpec((1, D), lambda i: (0, 0)),
            pl.BlockSpec((D, D), lambda i: (0, 0)),
            pl.BlockSpec((1, D), lambda i: (0, 0)),
        ],
        out_specs=pl.BlockSpec((BM, D), lambda i: (i, 0)),
        out_shape=jax.ShapeDtypeStruct((N, D), jnp.float32),
    )(part, selfv, w3, b3, w4, b4)


@jax.jit
def kernel(x, edge_index, emb, Wl1, Wr1, Wl2, Wr2, W3, b3, W4, b4):
    del x
    ei = edge_index.astype(jnp.int32).reshape(2 * E)
    b3r = b3.reshape(1, D)
    b4r = b4.reshape(1, D)

    ht1 = _mm(emb, Wr1)
    part1 = _sc_agg(ht1, ei)
    self1 = _mm(emb, Wl1)
    h1, ht2 = _k2(part1, self1, Wr2)
    part2 = _sc_agg(ht2, ei)
    self2 = _mm(h1, Wl2)
    return _k3(part2, self2, W3, b3r, W4, b4r)
